# baseline (speedup 1.0000x reference)
import functools

import jax
import jax.numpy as jnp
from jax import lax
from jax.experimental import pallas as pl
from jax.experimental.pallas import tpu as pltpu
from jax.experimental.pallas import tpu_sc as plsc

H = 96
BE = 256
NC, NS = 2, 16
CH = 128

NP_USER, NP_AP, NP_TGT = 10240, 2048, 4096
SH = 12288
STRIPE = SH // NS

SET_CORE = (0, 0, 1, 1, 1, 1)
SET_OFF = (0, NP_USER, 0, NP_AP, 2 * NP_AP, 2 * NP_AP + NP_TGT)


BN = 256


def _padw(w, rows=None, cols=None):
    r = (rows or w.shape[0]) - w.shape[0]
    c = (cols or w.shape[1]) - w.shape[1]
    return jnp.pad(w, ((0, r), (0, c)))


def _enc_body(x_ref, w1_ref, b1_ref, w2_ref, b2_ref, g_ref, bb_ref, o_ref):
    h1 = jnp.maximum(
        jnp.dot(x_ref[...], w1_ref[...], preferred_element_type=jnp.float32)
        + b1_ref[...], 0.0)
    h2 = (jnp.dot(h1, w2_ref[...], preferred_element_type=jnp.float32)
          + b2_ref[...])
    m = jnp.sum(h2, axis=1, keepdims=True) / H
    v = jnp.sum(h2 * h2, axis=1, keepdims=True) / H - m * m
    o_ref[...] = (h2 - m) * jax.lax.rsqrt(v + 1e-5) * g_ref[...] + bb_ref[...]


def _encode(x, p, ln, n_pad):
    d = x.shape[1]
    x = jnp.pad(x, ((0, n_pad - x.shape[0]), (0, 0)))
    w1 = p["l1"]["w"]
    b1 = _padw(p["l1"]["b"].reshape(1, H), cols=HP)
    w2 = _padw(p["l2"]["w"], rows=HP, cols=HP)
    b2 = _padw(p["l2"]["b"].reshape(1, H), cols=HP)
    g = _padw(ln["g"].reshape(1, H), cols=HP)
    bb = _padw(ln["b"].reshape(1, H), cols=HP)
    w1 = jnp.pad(w1, ((0, 0), (0, HP - H)))
    return pl.pallas_call(
        _enc_body,
        grid=(n_pad // BN,),
        in_specs=[
            pl.BlockSpec((BN, d), lambda i: (i, 0)),
            pl.BlockSpec((d, HP), lambda i: (0, 0)),
            pl.BlockSpec((1, HP), lambda i: (0, 0)),
            pl.BlockSpec((HP, HP), lambda i: (0, 0)),
            pl.BlockSpec((1, HP), lambda i: (0, 0)),
            pl.BlockSpec((1, HP), lambda i: (0, 0)),
            pl.BlockSpec((1, HP), lambda i: (0, 0)),
        ],
        out_specs=pl.BlockSpec((BN, HP), lambda i: (i, 0)),
        out_shape=jax.ShapeDtypeStruct((n_pad, HP), jnp.float32),
    )(x, w1, b1, w2, b2, g, bb)


def _combine_body(nset, h_ref, r_ref, b_ref, *rest):
    out_ref = rest[-1]
    acc = (jnp.dot(h_ref[...], r_ref[...],
                   preferred_element_type=jnp.float32) + b_ref[...])
    for k in range(nset):
        acc = acc + rest[2 * k][0] * rest[2 * k + 1][0]
    out_ref[...] = jnp.maximum(acc, 0.0)


def _combine(h, roots, biases, p, inv, set_ids, n_pad):
    nset = len(set_ids)
    r = _padw(sum(roots), rows=HP, cols=HP)
    b = _padw(sum(biases).reshape(1, H), cols=HP)
    specs = [
        pl.BlockSpec((BN, HP), lambda i: (i, 0)),
        pl.BlockSpec((HP, HP), lambda i: (0, 0)),
        pl.BlockSpec((1, HP), lambda i: (0, 0)),
    ]
    args = [h, r, b]
    for k in set_ids:
        core, off = SET_CORE[k], SET_OFF[k]
        idx_map = functools.partial(
            lambda c, o, i: (c, o + i, 0), core, off // BN)
        specs.append(pl.BlockSpec((1, BN, HP), idx_map))
        specs.append(pl.BlockSpec((1, BN, HP), idx_map))
        args.append(p)
        args.append(inv)
    return pl.pallas_call(
        functools.partial(_combine_body, nset),
        grid=(n_pad // BN,),
        in_specs=specs,
        out_specs=pl.BlockSpec((BN, HP), lambda i: (i, 0)),
        out_shape=jax.ShapeDtypeStruct((n_pad, HP), jnp.float32),
    )(*args)


def _head_body(x1_ref, x2_ref, x3_ref, w1a_ref, w1b_ref, w1c_ref, b1_ref,
               w2_ref, b2_ref, o_ref):
    h1 = (jnp.dot(x1_ref[...], w1a_ref[...],
                  preferred_element_type=jnp.float32) + b1_ref[...])
    if x2_ref is not None:
        h1 = h1 + jnp.dot(x2_ref[...], w1b_ref[...],
                          preferred_element_type=jnp.float32)
        h1 = h1 + jnp.dot(x3_ref[...], w1c_ref[...],
                          preferred_element_type=jnp.float32)
    h1 = jnp.maximum(h1, 0.0)
    o_ref[...] = (jnp.dot(h1, w2_ref[...],
                          preferred_element_type=jnp.float32) + b2_ref[...])


def _head(p, x1, x2=None, x3=None):
    n_pad = x1.shape[0]
    w1 = p["l1"]["w"]
    d1 = x1.shape[1]
    if x2 is None:
        w1a = _padw(w1, rows=d1, cols=HP)
        body = lambda x1r, w1ar, b1r, w2r, b2r, outr: _head_body(
            x1r, None, None, w1ar, None, None, b1r, w2r, b2r, outr)
        extra_specs, extra_args = [], []
    else:
        w1a = _padw(w1[:H], rows=d1, cols=HP)
        w1b = _padw(w1[H:2 * H], rows=x2.shape[1], cols=HP)
        w1c = _padw(w1[2 * H:], cols=HP)
        body = _head_body
        extra_specs = [
            pl.BlockSpec((BN, x2.shape[1]), lambda i: (i, 0)),
            pl.BlockSpec((BN, x3.shape[1]), lambda i: (i, 0)),
        ]
        extra_args = [x2, x3]
    b1 = _padw(p["l1"]["b"].reshape(1, H), cols=HP)
    w2 = _padw(p["l2"]["w"], rows=HP, cols=HP)
    b2 = _padw(p["l2"]["b"].reshape(1, 1), cols=HP)
    wspecs = ([pl.BlockSpec((d1, HP), lambda i: (0, 0))]
              + ([] if x2 is None else
                 [pl.BlockSpec((x2.shape[1], HP), lambda i: (0, 0)),
                  pl.BlockSpec((x3.shape[1], HP), lambda i: (0, 0))])
              + [pl.BlockSpec((1, HP), lambda i: (0, 0)),
                 pl.BlockSpec((HP, HP), lambda i: (0, 0)),
                 pl.BlockSpec((1, HP), lambda i: (0, 0))])
    wargs = ([w1a] + ([] if x2 is None else [w1b, w1c]) + [b1, w2, b2])
    return pl.pallas_call(
        body,
        grid=(n_pad // BN,),
        in_specs=([pl.BlockSpec((BN, d1), lambda i: (i, 0))] + extra_specs
                  + wspecs),
        out_specs=pl.BlockSpec((BN, HP), lambda i: (i, 0)),
        out_shape=jax.ShapeDtypeStruct((n_pad, HP), jnp.float32),
    )(x1, *extra_args, *wargs)


HP = 128


def _msg_body(e_real, ea_ref, xf_ref, xr_ref, w1_ref, b1_ref, w2_ref, b2_ref,
              of_ref, or_ref):
    g = jnp.maximum(
        jnp.dot(ea_ref[...], w1_ref[...], preferred_element_type=jnp.float32)
        + b1_ref[...], 0.0)
    w = (jnp.dot(g.astype(jnp.bfloat16), w2_ref[...],
                 preferred_element_type=jnp.float32)
         + b2_ref[...]).astype(jnp.bfloat16)
    w3 = w.reshape(BE, H, HP)
    rows = pl.program_id(0) * BE + lax.broadcasted_iota(jnp.int32, (BE, 1), 0)
    valid = rows < e_real
    mf = jnp.einsum('eh,eho->eo',
                    xf_ref[...][:, :H].astype(jnp.bfloat16), w3,
                    preferred_element_type=jnp.float32)
    mr = jnp.einsum('eh,eho->eo',
                    xr_ref[...][:, :H].astype(jnp.bfloat16), w3,
                    preferred_element_type=jnp.float32)
    of_ref[...] = jnp.where(valid, mf, 0.0)
    or_ref[...] = jnp.where(valid, mr, 0.0)


def _fused_msgs(ea, xf, xr, net, e_real):
    e_pad = ea.shape[0]
    de = ea.shape[1]
    w1 = net["l1"]["w"]
    b1 = net["l1"]["b"].reshape(1, H)
    w2p = jnp.pad(net["l2"]["w"].reshape(H, H, H),
                  ((0, 0), (0, 0), (0, HP - H))).reshape(H, H * HP)
    b2p = jnp.pad(net["l2"]["b"].reshape(H, H),
                  ((0, 0), (0, HP - H))).reshape(1, H * HP)
    grid = (e_pad // BE,)
    of, orv = pl.pallas_call(
        functools.partial(_msg_body, e_real),
        grid=grid,
        in_specs=[
            pl.BlockSpec((BE, de), lambda i: (i, 0)),
            pl.BlockSpec((BE, HP), lambda i: (i, 0)),
            pl.BlockSpec((BE, HP), lambda i: (i, 0)),
            pl.BlockSpec((de, H), lambda i: (0, 0)),
            pl.BlockSpec((1, H), lambda i: (0, 0)),
            pl.BlockSpec((H, H * HP), lambda i: (0, 0)),
            pl.BlockSpec((1, H * HP), lambda i: (0, 0)),
        ],
        out_specs=[
            pl.BlockSpec((BE, HP), lambda i: (i, 0)),
            pl.BlockSpec((BE, HP), lambda i: (i, 0)),
        ],
        out_shape=[
            jax.ShapeDtypeStruct((e_pad, HP), jnp.float32),
            jax.ShapeDtypeStruct((e_pad, HP), jnp.float32),
        ],
    )(ea, xf, xr, w1, b1, w2p.astype(jnp.bfloat16), b2p)
    return of, orv


def _gather_body(table_ids, nchs, *refs):
    ntab = max(table_ids) + 1
    nset = len(nchs)
    tables = refs[:ntab]
    idxs = refs[ntab:ntab + nset]
    outs = refs[ntab + nset:ntab + 2 * nset]
    vbuf, ibuf, sem = refs[-3], refs[-2], refs[-1]
    cid = lax.axis_index("c")
    sid = lax.axis_index("s")
    wid = sid * NC + cid
    nw = NC * NS
    for k in range(nset):
        nch = nchs[k]
        for j in range(-(-nch // nw)):
            c = wid + nw * j

            @pl.when(c < nch)
            def _():
                pltpu.sync_copy(idxs[k].at[c], ibuf)
                pltpu.async_copy(tables[table_ids[k]].at[ibuf], vbuf,
                                 sem).wait()
                pltpu.sync_copy(vbuf, outs[k].at[pl.ds(c * CH, CH)])


def _sc_gather(tables, table_ids, idx2ds):
    nchs = tuple(ix.shape[0] for ix in idx2ds)
    mesh = plsc.VectorSubcoreMesh(core_axis_name="c", subcore_axis_name="s",
                                  num_cores=NC, num_subcores=NS)
    outs = pl.kernel(
        functools.partial(_gather_body, table_ids, nchs),
        out_type=[jax.ShapeDtypeStruct((n * CH, HP), jnp.float32)
                  for n in nchs],
        mesh=mesh,
        scratch_types=[
            pltpu.VMEM((CH, HP), jnp.float32),
            pltpu.VMEM((CH,), jnp.int32),
            pltpu.SemaphoreType.DMA,
        ],
    )(*tables, *idx2ds)
    return outs


def _scatter_body(eps, width, *refs):
    msgs = refs[0:6]
    idxs = refs[6:12]
    zeros = refs[12]
    out = refs[13]
    vbuf = refs[14]
    ibuf = refs[15]
    sh = refs[16]
    cid = lax.axis_index("c")
    sid = lax.axis_index("s")
    pltpu.sync_copy(zeros, sh.at[pl.ds(sid * STRIPE, STRIPE)])
    plsc.subcore_barrier()
    for k in range(6):
        nch = eps[k] // CH
        for j in range(-(-nch // NS)):
            c = sid + NS * j

            @pl.when(jnp.logical_and(cid == SET_CORE[k], c < nch))
            def _():
                pltpu.sync_copy(idxs[k].at[c], ibuf)
                pltpu.sync_copy(msgs[k].at[pl.ds(c * CH, CH)], vbuf)
                pltpu.sync_copy(vbuf, sh.at[ibuf], add=True)
    plsc.subcore_barrier()
    pltpu.sync_copy(sh.at[pl.ds(sid * STRIPE, STRIPE)],
                    out.at[cid].at[pl.ds(sid * STRIPE, STRIPE)])


def _sc_scatter(msgs, idx2ds, width):
    eps = tuple(m.shape[0] for m in msgs)
    mesh = plsc.VectorSubcoreMesh(core_axis_name="c", subcore_axis_name="s",
                                  num_cores=NC, num_subcores=NS)
    zeros = jnp.zeros((STRIPE, width), jnp.float32)
    out = pl.kernel(
        functools.partial(_scatter_body, eps, width),
        out_type=jax.ShapeDtypeStruct((NC, SH, width), jnp.float32),
        mesh=mesh,
        scratch_types=[
            pltpu.VMEM((CH, width), jnp.float32),
            pltpu.VMEM((CH,), jnp.int32),
            pltpu.VMEM_SHARED((SH, width), jnp.float32),
        ],
    )(*msgs, *idx2ds, zeros)
    return out


def _prep_idx(idx, e_pad, off):
    idx = jnp.pad(idx, (0, e_pad - idx.shape[0])) + off
    return idx.reshape(e_pad // CH, CH).astype(jnp.int32)


def _pad_rows(x, e_pad):
    return jnp.pad(x, ((0, e_pad - x.shape[0]), (0, 0)))


def kernel(x_ap, x_user, x_target, ea_s, ea_tx, ea_rx, params, ei_s, ei_tx,
           ei_rx):
    h_ap = _encode(x_ap, params["ap_in"], params["ln"]["ap"], NP_AP)
    h_user = _encode(x_user, params["user_in"], params["ln"]["user"], NP_USER)
    h_tgt = _encode(x_target, params["tgt_in"], params["ln"]["tgt"], NP_TGT)

    e_s, e_tx, e_rx = ea_s.shape[0], ea_tx.shape[0], ea_rx.shape[0]
    ep_s = (e_s + BE - 1) // BE * BE
    ep_tx = (e_tx + BE - 1) // BE * BE
    ep_rx = (e_rx + BE - 1) // BE * BE
    ea_s_p = _pad_rows(ea_s, ep_s)
    ea_tx_p = _pad_rows(ea_tx, ep_tx)
    ea_rx_p = _pad_rows(ea_rx, ep_rx)

    n_ap, n_user, n_tgt = x_ap.shape[0], x_user.shape[0], x_target.shape[0]

    idx2ds = (
        _prep_idx(ei_s[1], ep_s, SET_OFF[0]),
        _prep_idx(ei_s[0], ep_s, SET_OFF[1]),
        _prep_idx(ei_tx[0], ep_tx, SET_OFF[2]),
        _prep_idx(ei_rx[0], ep_rx, SET_OFF[3]),
        _prep_idx(ei_tx[1], ep_tx, SET_OFF[4]),
        _prep_idx(ei_rx[1], ep_rx, SET_OFF[5]),
    )

    def ones_masked(e_pad, e_real):
        return jnp.where(jnp.arange(e_pad)[:, None] < e_real,
                         jnp.float32(1), jnp.float32(0)) * jnp.ones((1, HP),
                                                                    jnp.float32)
    cnt = _sc_scatter(
        (ones_masked(ep_s, e_s), ones_masked(ep_s, e_s),
         ones_masked(ep_tx, e_tx), ones_masked(ep_rx, e_rx),
         ones_masked(ep_tx, e_tx), ones_masked(ep_rx, e_rx)),
        idx2ds, HP)
    inv = 1.0 / jnp.maximum(cnt, 1.0)

    gidx = (
        _prep_idx(ei_s[0], ep_s, 0), _prep_idx(ei_s[1], ep_s, 0),
        _prep_idx(ei_tx[0], ep_tx, 0), _prep_idx(ei_tx[1], ep_tx, 0),
        _prep_idx(ei_rx[0], ep_rx, 0), _prep_idx(ei_rx[1], ep_rx, 0),
    )

    for lp in params["layers"]:
        c = lp["conv"]
        (xs_ap, xs_user, xtx_ap, xtx_tgt, xrx_ap, xrx_tgt) = _sc_gather(
            (h_ap, h_user, h_tgt), (0, 1, 0, 2, 0, 2), gidx)

        m_s_f, m_s_r = _fused_msgs(ea_s_p, xs_ap, xs_user, lp["net_serv"], e_s)
        m_tx_f, m_tx_r = _fused_msgs(ea_tx_p, xtx_ap, xtx_tgt,
                                     lp["net_sens"], e_tx)
        m_rx_f, m_rx_r = _fused_msgs(ea_rx_p, xrx_ap, xrx_tgt,
                                     lp["net_sens"], e_rx)

        p = _sc_scatter((m_s_f, m_s_r, m_tx_r, m_rx_r, m_tx_f, m_rx_f),
                        idx2ds, HP)
        h_user = _combine(h_user, [c["serves"]["root"]],
                          [c["serves"]["bias"]], p, inv, (0,), NP_USER)
        h_ap = _combine(h_ap,
                        [c["rev_served"]["root"], c["rev_tx"]["root"],
                         c["rev_rx"]["root"]],
                        [c["rev_served"]["bias"], c["rev_tx"]["bias"],
                         c["rev_rx"]["bias"]], p, inv, (1, 2, 3), NP_AP)
        h_tgt = _combine(h_tgt, [c["tx"]["root"], c["rx"]["root"]],
                         [c["tx"]["bias"], c["rx"]["bias"]], p, inv, (4, 5),
                         NP_TGT)

    ga, gu = _sc_gather((h_ap, h_user), (0, 1), (gidx[0], gidx[1]))
    x_logit = _head(params["edge_head"], ga, gu, ea_s_p)[:e_s, :1]
    tau_logit = _head(params["ap_head"], h_ap)[:n_ap, :1]
    s_logit = _head(params["tgt_head"], h_tgt)[:n_tgt, :1]
    return x_logit, tau_logit, s_logit


# speedup vs baseline: 1.0036x; 1.0036x over previous
import functools

import jax
import jax.numpy as jnp
from jax import lax
from jax.experimental import pallas as pl
from jax.experimental.pallas import tpu as pltpu
from jax.experimental.pallas import tpu_sc as plsc

H = 96
BE = 256
NC, NS = 2, 16
CH = 128

NP_USER, NP_AP, NP_TGT = 10240, 2048, 4096
SH = 12288
STRIPE = SH // NS

SET_CORE = (0, 0, 1, 1, 1, 1)
SET_OFF = (0, NP_USER, 0, NP_AP, 2 * NP_AP, 2 * NP_AP + NP_TGT)


BN = 256


def _padw(w, rows=None, cols=None):
    r = (rows or w.shape[0]) - w.shape[0]
    c = (cols or w.shape[1]) - w.shape[1]
    return jnp.pad(w, ((0, r), (0, c)))


def _enc_body(x_ref, w1_ref, b1_ref, w2_ref, b2_ref, g_ref, bb_ref, o_ref):
    h1 = jnp.maximum(
        jnp.dot(x_ref[...], w1_ref[...], preferred_element_type=jnp.float32)
        + b1_ref[...], 0.0)
    h2 = (jnp.dot(h1, w2_ref[...], preferred_element_type=jnp.float32)
          + b2_ref[...])
    m = jnp.sum(h2, axis=1, keepdims=True) / H
    v = jnp.sum(h2 * h2, axis=1, keepdims=True) / H - m * m
    o_ref[...] = (h2 - m) * jax.lax.rsqrt(v + 1e-5) * g_ref[...] + bb_ref[...]


def _encode(x, p, ln, n_pad):
    d = x.shape[1]
    x = jnp.pad(x, ((0, n_pad - x.shape[0]), (0, 0)))
    w1 = p["l1"]["w"]
    b1 = _padw(p["l1"]["b"].reshape(1, H), cols=HP)
    w2 = _padw(p["l2"]["w"], rows=HP, cols=HP)
    b2 = _padw(p["l2"]["b"].reshape(1, H), cols=HP)
    g = _padw(ln["g"].reshape(1, H), cols=HP)
    bb = _padw(ln["b"].reshape(1, H), cols=HP)
    w1 = jnp.pad(w1, ((0, 0), (0, HP - H)))
    return pl.pallas_call(
        _enc_body,
        grid=(n_pad // BN,),
        in_specs=[
            pl.BlockSpec((BN, d), lambda i: (i, 0)),
            pl.BlockSpec((d, HP), lambda i: (0, 0)),
            pl.BlockSpec((1, HP), lambda i: (0, 0)),
            pl.BlockSpec((HP, HP), lambda i: (0, 0)),
            pl.BlockSpec((1, HP), lambda i: (0, 0)),
            pl.BlockSpec((1, HP), lambda i: (0, 0)),
            pl.BlockSpec((1, HP), lambda i: (0, 0)),
        ],
        out_specs=pl.BlockSpec((BN, HP), lambda i: (i, 0)),
        out_shape=jax.ShapeDtypeStruct((n_pad, HP), jnp.float32),
    )(x, w1, b1, w2, b2, g, bb)


def _combine_body(nset, h_ref, r_ref, b_ref, *rest):
    out_ref = rest[-1]
    acc = (jnp.dot(h_ref[...], r_ref[...],
                   preferred_element_type=jnp.float32) + b_ref[...])
    for k in range(nset):
        acc = acc + rest[2 * k][0] * rest[2 * k + 1][0]
    out_ref[...] = jnp.maximum(acc, 0.0)


def _combine(h, roots, biases, p, inv, set_ids, n_pad):
    nset = len(set_ids)
    r = _padw(sum(roots), rows=HP, cols=HP)
    b = _padw(sum(biases).reshape(1, H), cols=HP)
    specs = [
        pl.BlockSpec((BN, HP), lambda i: (i, 0)),
        pl.BlockSpec((HP, HP), lambda i: (0, 0)),
        pl.BlockSpec((1, HP), lambda i: (0, 0)),
    ]
    args = [h, r, b]
    for k in set_ids:
        core, off = SET_CORE[k], SET_OFF[k]
        idx_map = functools.partial(
            lambda c, o, i: (c, o + i, 0), core, off // BN)
        specs.append(pl.BlockSpec((1, BN, HP), idx_map))
        specs.append(pl.BlockSpec((1, BN, HP), idx_map))
        args.append(p)
        args.append(inv)
    return pl.pallas_call(
        functools.partial(_combine_body, nset),
        grid=(n_pad // BN,),
        in_specs=specs,
        out_specs=pl.BlockSpec((BN, HP), lambda i: (i, 0)),
        out_shape=jax.ShapeDtypeStruct((n_pad, HP), jnp.float32),
    )(*args)


def _head_body(x1_ref, x2_ref, x3_ref, w1a_ref, w1b_ref, w1c_ref, b1_ref,
               w2_ref, b2_ref, o_ref):
    h1 = (jnp.dot(x1_ref[...], w1a_ref[...],
                  preferred_element_type=jnp.float32) + b1_ref[...])
    if x2_ref is not None:
        h1 = h1 + jnp.dot(x2_ref[...], w1b_ref[...],
                          preferred_element_type=jnp.float32)
        h1 = h1 + jnp.dot(x3_ref[...], w1c_ref[...],
                          preferred_element_type=jnp.float32)
    h1 = jnp.maximum(h1, 0.0)
    o_ref[...] = (jnp.dot(h1, w2_ref[...],
                          preferred_element_type=jnp.float32) + b2_ref[...])


def _head(p, x1, x2=None, x3=None):
    n_pad = x1.shape[0]
    w1 = p["l1"]["w"]
    d1 = x1.shape[1]
    if x2 is None:
        w1a = _padw(w1, rows=d1, cols=HP)
        body = lambda x1r, w1ar, b1r, w2r, b2r, outr: _head_body(
            x1r, None, None, w1ar, None, None, b1r, w2r, b2r, outr)
        extra_specs, extra_args = [], []
    else:
        w1a = _padw(w1[:H], rows=d1, cols=HP)
        w1b = _padw(w1[H:2 * H], rows=x2.shape[1], cols=HP)
        w1c = _padw(w1[2 * H:], cols=HP)
        body = _head_body
        extra_specs = [
            pl.BlockSpec((BN, x2.shape[1]), lambda i: (i, 0)),
            pl.BlockSpec((BN, x3.shape[1]), lambda i: (i, 0)),
        ]
        extra_args = [x2, x3]
    b1 = _padw(p["l1"]["b"].reshape(1, H), cols=HP)
    w2 = _padw(p["l2"]["w"], rows=HP, cols=HP)
    b2 = _padw(p["l2"]["b"].reshape(1, 1), cols=HP)
    wspecs = ([pl.BlockSpec((d1, HP), lambda i: (0, 0))]
              + ([] if x2 is None else
                 [pl.BlockSpec((x2.shape[1], HP), lambda i: (0, 0)),
                  pl.BlockSpec((x3.shape[1], HP), lambda i: (0, 0))])
              + [pl.BlockSpec((1, HP), lambda i: (0, 0)),
                 pl.BlockSpec((HP, HP), lambda i: (0, 0)),
                 pl.BlockSpec((1, HP), lambda i: (0, 0))])
    wargs = ([w1a] + ([] if x2 is None else [w1b, w1c]) + [b1, w2, b2])
    return pl.pallas_call(
        body,
        grid=(n_pad // BN,),
        in_specs=([pl.BlockSpec((BN, d1), lambda i: (i, 0))] + extra_specs
                  + wspecs),
        out_specs=pl.BlockSpec((BN, HP), lambda i: (i, 0)),
        out_shape=jax.ShapeDtypeStruct((n_pad, HP), jnp.float32),
    )(x1, *extra_args, *wargs)


HP = 128


def _msg_body(e_real, ea_ref, xf_ref, xr_ref, w1_ref, b1_ref, w2_ref, b2_ref,
              of_ref, or_ref):
    g = jnp.maximum(
        jnp.dot(ea_ref[...], w1_ref[...], preferred_element_type=jnp.float32)
        + b1_ref[...], 0.0)
    w = (jnp.dot(g.astype(jnp.bfloat16), w2_ref[...],
                 preferred_element_type=jnp.float32)
         + b2_ref[...]).astype(jnp.bfloat16)
    w3 = w.reshape(BE, H, HP)
    rows = pl.program_id(0) * BE + lax.broadcasted_iota(jnp.int32, (BE, 1), 0)
    valid = rows < e_real
    mf = jnp.einsum('eh,eho->eo',
                    xf_ref[...][:, :H].astype(jnp.bfloat16), w3,
                    preferred_element_type=jnp.float32)
    mr = jnp.einsum('eh,eho->eo',
                    xr_ref[...][:, :H].astype(jnp.bfloat16), w3,
                    preferred_element_type=jnp.float32)
    of_ref[...] = jnp.where(valid, mf, 0.0)
    or_ref[...] = jnp.where(valid, mr, 0.0)


def _fused_msgs(ea, xf, xr, net, e_real):
    e_pad = ea.shape[0]
    de = ea.shape[1]
    w1 = net["l1"]["w"]
    b1 = net["l1"]["b"].reshape(1, H)
    w2p = jnp.pad(net["l2"]["w"].reshape(H, H, H),
                  ((0, 0), (0, 0), (0, HP - H))).reshape(H, H * HP)
    b2p = jnp.pad(net["l2"]["b"].reshape(H, H),
                  ((0, 0), (0, HP - H))).reshape(1, H * HP)
    grid = (e_pad // BE,)
    of, orv = pl.pallas_call(
        functools.partial(_msg_body, e_real),
        grid=grid,
        in_specs=[
            pl.BlockSpec((BE, de), lambda i: (i, 0)),
            pl.BlockSpec((BE, HP), lambda i: (i, 0)),
            pl.BlockSpec((BE, HP), lambda i: (i, 0)),
            pl.BlockSpec((de, H), lambda i: (0, 0)),
            pl.BlockSpec((1, H), lambda i: (0, 0)),
            pl.BlockSpec((H, H * HP), lambda i: (0, 0)),
            pl.BlockSpec((1, H * HP), lambda i: (0, 0)),
        ],
        out_specs=[
            pl.BlockSpec((BE, HP), lambda i: (i, 0)),
            pl.BlockSpec((BE, HP), lambda i: (i, 0)),
        ],
        out_shape=[
            jax.ShapeDtypeStruct((e_pad, HP), jnp.float32),
            jax.ShapeDtypeStruct((e_pad, HP), jnp.float32),
        ],
    )(ea, xf, xr, w1, b1, w2p.astype(jnp.bfloat16), b2p)
    return of, orv


def _gather_body(table_ids, rpws, uniq, *refs):
    ntab = max(table_ids) + 1
    nset = len(rpws)
    tables = refs[:ntab]
    idxs = refs[ntab:ntab + nset]
    outs = refs[ntab + nset:ntab + 2 * nset]
    nu = len(uniq)
    vbuf = refs[ntab + 2 * nset]
    ibufs = refs[ntab + 2 * nset + 1:ntab + 2 * nset + 1 + nu]
    sem = refs[-1]
    cid = lax.axis_index("c")
    sid = lax.axis_index("s")
    wid = sid * NC + cid
    for k in range(nset):
        rpw = rpws[k]
        ib = ibufs[uniq.index(rpw)]
        base = wid * rpw
        pltpu.sync_copy(idxs[k].at[pl.ds(base, rpw)], ib)
        pltpu.async_copy(tables[table_ids[k]].at[ib],
                         vbuf.at[pl.ds(0, rpw)], sem).wait()
        pltpu.sync_copy(vbuf.at[pl.ds(0, rpw)], outs[k].at[pl.ds(base, rpw)])


def _sc_gather(tables, table_ids, idxs):
    rpws = tuple(ix.shape[0] // (NC * NS) for ix in idxs)
    uniq = sorted(set(rpws), reverse=True)
    mesh = plsc.VectorSubcoreMesh(core_axis_name="c", subcore_axis_name="s",
                                  num_cores=NC, num_subcores=NS)
    outs = pl.kernel(
        functools.partial(_gather_body, table_ids, rpws, uniq),
        out_type=[jax.ShapeDtypeStruct((ix.shape[0], HP), jnp.float32)
                  for ix in idxs],
        mesh=mesh,
        scratch_types=(
            [pltpu.VMEM((uniq[0], HP), jnp.float32)]
            + [pltpu.VMEM((r,), jnp.int32) for r in uniq]
            + [pltpu.SemaphoreType.DMA]),
    )(*tables, *idxs)
    return outs


def _scatter_body(eps, *refs):
    msgs = refs[0:6]
    idxs = refs[6:12]
    zeros = refs[12]
    out = refs[13]
    vbuf = refs[14]
    ibuf = refs[15]
    sh = refs[16]
    cid = lax.axis_index("c")
    sid = lax.axis_index("s")
    pltpu.sync_copy(zeros, sh.at[pl.ds(sid * STRIPE, STRIPE)])
    plsc.subcore_barrier()
    for k in range(6):
        nch = eps[k] // CH
        for j in range(-(-nch // NS)):
            c = sid + NS * j

            @pl.when(jnp.logical_and(cid == SET_CORE[k], c < nch))
            def _():
                pltpu.sync_copy(idxs[k].at[c], ibuf)
                pltpu.sync_copy(msgs[k].at[pl.ds(c * CH, CH)], vbuf)
                pltpu.sync_copy(vbuf, sh.at[ibuf], add=True)
    plsc.subcore_barrier()
    pltpu.sync_copy(sh.at[pl.ds(sid * STRIPE, STRIPE)],
                    out.at[cid].at[pl.ds(sid * STRIPE, STRIPE)])


def _sc_scatter(msgs, idxs, width):
    eps = tuple(m.shape[0] for m in msgs)
    mesh = plsc.VectorSubcoreMesh(core_axis_name="c", subcore_axis_name="s",
                                  num_cores=NC, num_subcores=NS)
    zeros = jnp.zeros((STRIPE, width), jnp.float32)
    out = pl.kernel(
        functools.partial(_scatter_body, eps),
        out_type=jax.ShapeDtypeStruct((NC, SH, width), jnp.float32),
        mesh=mesh,
        scratch_types=[
            pltpu.VMEM((CH, width), jnp.float32),
            pltpu.VMEM((CH,), jnp.int32),
            pltpu.VMEM_SHARED((SH, width), jnp.float32),
        ],
    )(*msgs, *[ix.reshape(-1, CH) for ix in idxs], zeros)
    return out


def _prep_idx(idx, e_pad, off):
    return (jnp.pad(idx, (0, e_pad - idx.shape[0])) + off).astype(jnp.int32)


def _pad_rows(x, e_pad):
    return jnp.pad(x, ((0, e_pad - x.shape[0]), (0, 0)))


def kernel(x_ap, x_user, x_target, ea_s, ea_tx, ea_rx, params, ei_s, ei_tx,
           ei_rx):
    h_ap = _encode(x_ap, params["ap_in"], params["ln"]["ap"], NP_AP)
    h_user = _encode(x_user, params["user_in"], params["ln"]["user"], NP_USER)
    h_tgt = _encode(x_target, params["tgt_in"], params["ln"]["tgt"], NP_TGT)

    e_s, e_tx, e_rx = ea_s.shape[0], ea_tx.shape[0], ea_rx.shape[0]
    ep_s = (e_s + BE - 1) // BE * BE
    ep_tx = (e_tx + BE - 1) // BE * BE
    ep_rx = (e_rx + BE - 1) // BE * BE
    ea_s_p = _pad_rows(ea_s, ep_s)
    ea_tx_p = _pad_rows(ea_tx, ep_tx)
    ea_rx_p = _pad_rows(ea_rx, ep_rx)

    n_ap, n_user, n_tgt = x_ap.shape[0], x_user.shape[0], x_target.shape[0]

    idx2ds = (
        _prep_idx(ei_s[1], ep_s, SET_OFF[0]),
        _prep_idx(ei_s[0], ep_s, SET_OFF[1]),
        _prep_idx(ei_tx[0], ep_tx, SET_OFF[2]),
        _prep_idx(ei_rx[0], ep_rx, SET_OFF[3]),
        _prep_idx(ei_tx[1], ep_tx, SET_OFF[4]),
        _prep_idx(ei_rx[1], ep_rx, SET_OFF[5]),
    )

    def ones_masked(e_pad, e_real):
        return jnp.where(jnp.arange(e_pad)[:, None] < e_real,
                         jnp.float32(1), jnp.float32(0)) * jnp.ones((1, HP),
                                                                    jnp.float32)
    cnt = _sc_scatter(
        (ones_masked(ep_s, e_s), ones_masked(ep_s, e_s),
         ones_masked(ep_tx, e_tx), ones_masked(ep_rx, e_rx),
         ones_masked(ep_tx, e_tx), ones_masked(ep_rx, e_rx)),
        idx2ds, HP)
    inv = 1.0 / jnp.maximum(cnt, 1.0)

    gidx = (
        _prep_idx(ei_s[0], ep_s, 0), _prep_idx(ei_s[1], ep_s, 0),
        _prep_idx(ei_tx[0], ep_tx, 0), _prep_idx(ei_tx[1], ep_tx, 0),
        _prep_idx(ei_rx[0], ep_rx, 0), _prep_idx(ei_rx[1], ep_rx, 0),
    )

    for lp in params["layers"]:
        c = lp["conv"]
        (xs_ap, xs_user, xtx_ap, xtx_tgt, xrx_ap, xrx_tgt) = _sc_gather(
            (h_ap, h_user, h_tgt), (0, 1, 0, 2, 0, 2), gidx)

        m_s_f, m_s_r = _fused_msgs(ea_s_p, xs_ap, xs_user, lp["net_serv"], e_s)
        m_tx_f, m_tx_r = _fused_msgs(ea_tx_p, xtx_ap, xtx_tgt,
                                     lp["net_sens"], e_tx)
        m_rx_f, m_rx_r = _fused_msgs(ea_rx_p, xrx_ap, xrx_tgt,
                                     lp["net_sens"], e_rx)

        p = _sc_scatter((m_s_f, m_s_r, m_tx_r, m_rx_r, m_tx_f, m_rx_f),
                        idx2ds, HP)
        h_user = _combine(h_user, [c["serves"]["root"]],
                          [c["serves"]["bias"]], p, inv, (0,), NP_USER)
        h_ap = _combine(h_ap,
                        [c["rev_served"]["root"], c["rev_tx"]["root"],
                         c["rev_rx"]["root"]],
                        [c["rev_served"]["bias"], c["rev_tx"]["bias"],
                         c["rev_rx"]["bias"]], p, inv, (1, 2, 3), NP_AP)
        h_tgt = _combine(h_tgt, [c["tx"]["root"], c["rx"]["root"]],
                         [c["tx"]["bias"], c["rx"]["bias"]], p, inv, (4, 5),
                         NP_TGT)

    ga, gu = _sc_gather((h_ap, h_user), (0, 1), (gidx[0], gidx[1]))
    x_logit = _head(params["edge_head"], ga, gu, ea_s_p)[:e_s, :1]
    tau_logit = _head(params["ap_head"], h_ap)[:n_ap, :1]
    s_logit = _head(params["tgt_head"], h_tgt)[:n_tgt, :1]
    return x_logit, tau_logit, s_logit


# speedup vs baseline: 1.0157x; 1.0121x over previous
import functools

import jax
import jax.numpy as jnp
from jax import lax
from jax.experimental import pallas as pl
from jax.experimental.pallas import tpu as pltpu
from jax.experimental.pallas import tpu_sc as plsc

H = 96
BE = 512
NC, NS = 2, 16
CH = 128

NP_USER, NP_AP, NP_TGT = 10240, 2048, 4096
SH = 12288
STRIPE = SH // NS

SET_CORE = (0, 0, 1, 1, 1, 1)
SET_OFF = (0, NP_USER, 0, NP_AP, 2 * NP_AP, 2 * NP_AP + NP_TGT)


BN = 256


def _padw(w, rows=None, cols=None):
    r = (rows or w.shape[0]) - w.shape[0]
    c = (cols or w.shape[1]) - w.shape[1]
    return jnp.pad(w, ((0, r), (0, c)))


def _enc_body(x_ref, w1_ref, b1_ref, w2_ref, b2_ref, g_ref, bb_ref, o_ref):
    h1 = jnp.maximum(
        jnp.dot(x_ref[...], w1_ref[...], preferred_element_type=jnp.float32)
        + b1_ref[...], 0.0)
    h2 = (jnp.dot(h1, w2_ref[...], preferred_element_type=jnp.float32)
          + b2_ref[...])
    m = jnp.sum(h2, axis=1, keepdims=True) / H
    v = jnp.sum(h2 * h2, axis=1, keepdims=True) / H - m * m
    o_ref[...] = (h2 - m) * jax.lax.rsqrt(v + 1e-5) * g_ref[...] + bb_ref[...]


def _encode(x, p, ln, n_pad):
    d = x.shape[1]
    x = jnp.pad(x, ((0, n_pad - x.shape[0]), (0, 0)))
    w1 = p["l1"]["w"]
    b1 = _padw(p["l1"]["b"].reshape(1, H), cols=HP)
    w2 = _padw(p["l2"]["w"], rows=HP, cols=HP)
    b2 = _padw(p["l2"]["b"].reshape(1, H), cols=HP)
    g = _padw(ln["g"].reshape(1, H), cols=HP)
    bb = _padw(ln["b"].reshape(1, H), cols=HP)
    w1 = jnp.pad(w1, ((0, 0), (0, HP - H)))
    return pl.pallas_call(
        _enc_body,
        grid=(n_pad // BN,),
        in_specs=[
            pl.BlockSpec((BN, d), lambda i: (i, 0)),
            pl.BlockSpec((d, HP), lambda i: (0, 0)),
            pl.BlockSpec((1, HP), lambda i: (0, 0)),
            pl.BlockSpec((HP, HP), lambda i: (0, 0)),
            pl.BlockSpec((1, HP), lambda i: (0, 0)),
            pl.BlockSpec((1, HP), lambda i: (0, 0)),
            pl.BlockSpec((1, HP), lambda i: (0, 0)),
        ],
        out_specs=pl.BlockSpec((BN, HP), lambda i: (i, 0)),
        out_shape=jax.ShapeDtypeStruct((n_pad, HP), jnp.float32),
    )(x, w1, b1, w2, b2, g, bb)


def _combine_body(nset, h_ref, r_ref, b_ref, *rest):
    out_ref = rest[-1]
    acc = (jnp.dot(h_ref[...], r_ref[...],
                   preferred_element_type=jnp.float32) + b_ref[...])
    for k in range(nset):
        acc = acc + rest[2 * k][0] * rest[2 * k + 1][0]
    out_ref[...] = jnp.maximum(acc, 0.0)


def _combine(h, roots, biases, p, inv, set_ids, n_pad):
    nset = len(set_ids)
    r = _padw(sum(roots), rows=HP, cols=HP)
    b = _padw(sum(biases).reshape(1, H), cols=HP)
    specs = [
        pl.BlockSpec((BN, HP), lambda i: (i, 0)),
        pl.BlockSpec((HP, HP), lambda i: (0, 0)),
        pl.BlockSpec((1, HP), lambda i: (0, 0)),
    ]
    args = [h, r, b]
    for k in set_ids:
        core, off = SET_CORE[k], SET_OFF[k]
        idx_map = functools.partial(
            lambda c, o, i: (c, o + i, 0), core, off // BN)
        specs.append(pl.BlockSpec((1, BN, HP), idx_map))
        specs.append(pl.BlockSpec((1, BN, HP), idx_map))
        args.append(p)
        args.append(inv)
    return pl.pallas_call(
        functools.partial(_combine_body, nset),
        grid=(n_pad // BN,),
        in_specs=specs,
        out_specs=pl.BlockSpec((BN, HP), lambda i: (i, 0)),
        out_shape=jax.ShapeDtypeStruct((n_pad, HP), jnp.float32),
    )(*args)


def _head_body(x1_ref, x2_ref, x3_ref, w1a_ref, w1b_ref, w1c_ref, b1_ref,
               w2_ref, b2_ref, o_ref):
    h1 = (jnp.dot(x1_ref[...], w1a_ref[...],
                  preferred_element_type=jnp.float32) + b1_ref[...])
    if x2_ref is not None:
        h1 = h1 + jnp.dot(x2_ref[...], w1b_ref[...],
                          preferred_element_type=jnp.float32)
        h1 = h1 + jnp.dot(x3_ref[...], w1c_ref[...],
                          preferred_element_type=jnp.float32)
    h1 = jnp.maximum(h1, 0.0)
    o_ref[...] = (jnp.dot(h1, w2_ref[...],
                          preferred_element_type=jnp.float32) + b2_ref[...])


def _head(p, x1, x2=None, x3=None):
    n_pad = x1.shape[0]
    w1 = p["l1"]["w"]
    d1 = x1.shape[1]
    if x2 is None:
        w1a = _padw(w1, rows=d1, cols=HP)
        body = lambda x1r, w1ar, b1r, w2r, b2r, outr: _head_body(
            x1r, None, None, w1ar, None, None, b1r, w2r, b2r, outr)
        extra_specs, extra_args = [], []
    else:
        w1a = _padw(w1[:H], rows=d1, cols=HP)
        w1b = _padw(w1[H:2 * H], rows=x2.shape[1], cols=HP)
        w1c = _padw(w1[2 * H:], cols=HP)
        body = _head_body
        extra_specs = [
            pl.BlockSpec((BN, x2.shape[1]), lambda i: (i, 0)),
            pl.BlockSpec((BN, x3.shape[1]), lambda i: (i, 0)),
        ]
        extra_args = [x2, x3]
    b1 = _padw(p["l1"]["b"].reshape(1, H), cols=HP)
    w2 = _padw(p["l2"]["w"], rows=HP, cols=HP)
    b2 = _padw(p["l2"]["b"].reshape(1, 1), cols=HP)
    wspecs = ([pl.BlockSpec((d1, HP), lambda i: (0, 0))]
              + ([] if x2 is None else
                 [pl.BlockSpec((x2.shape[1], HP), lambda i: (0, 0)),
                  pl.BlockSpec((x3.shape[1], HP), lambda i: (0, 0))])
              + [pl.BlockSpec((1, HP), lambda i: (0, 0)),
                 pl.BlockSpec((HP, HP), lambda i: (0, 0)),
                 pl.BlockSpec((1, HP), lambda i: (0, 0))])
    wargs = ([w1a] + ([] if x2 is None else [w1b, w1c]) + [b1, w2, b2])
    return pl.pallas_call(
        body,
        grid=(n_pad // BN,),
        in_specs=([pl.BlockSpec((BN, d1), lambda i: (i, 0))] + extra_specs
                  + wspecs),
        out_specs=pl.BlockSpec((BN, HP), lambda i: (i, 0)),
        out_shape=jax.ShapeDtypeStruct((n_pad, HP), jnp.float32),
    )(x1, *extra_args, *wargs)


HP = 128


def _msg_body(e_real, ea_ref, xf_ref, xr_ref, w1_ref, b1_ref, w2_ref, b2_ref,
              of_ref, or_ref):
    g = jnp.maximum(
        jnp.dot(ea_ref[...], w1_ref[...], preferred_element_type=jnp.float32)
        + b1_ref[...], 0.0)
    w = (jnp.dot(g.astype(jnp.bfloat16), w2_ref[...],
                 preferred_element_type=jnp.float32)
         + b2_ref[...]).astype(jnp.bfloat16)
    w3 = w.reshape(BE, H, HP)
    rows = pl.program_id(0) * BE + lax.broadcasted_iota(jnp.int32, (BE, 1), 0)
    valid = rows < e_real
    mf = jnp.einsum('eh,eho->eo',
                    xf_ref[...][:, :H].astype(jnp.bfloat16), w3,
                    preferred_element_type=jnp.float32)
    mr = jnp.einsum('eh,eho->eo',
                    xr_ref[...][:, :H].astype(jnp.bfloat16), w3,
                    preferred_element_type=jnp.float32)
    of_ref[...] = jnp.where(valid, mf, 0.0)
    or_ref[...] = jnp.where(valid, mr, 0.0)


def _fused_msgs(ea, xf, xr, net, e_real):
    e_pad = ea.shape[0]
    de = ea.shape[1]
    w1 = net["l1"]["w"]
    b1 = net["l1"]["b"].reshape(1, H)
    w2p = jnp.pad(net["l2"]["w"].reshape(H, H, H),
                  ((0, 0), (0, 0), (0, HP - H))).reshape(H, H * HP)
    b2p = jnp.pad(net["l2"]["b"].reshape(H, H),
                  ((0, 0), (0, HP - H))).reshape(1, H * HP)
    grid = (e_pad // BE,)
    of, orv = pl.pallas_call(
        functools.partial(_msg_body, e_real),
        grid=grid,
        in_specs=[
            pl.BlockSpec((BE, de), lambda i: (i, 0)),
            pl.BlockSpec((BE, HP), lambda i: (i, 0)),
            pl.BlockSpec((BE, HP), lambda i: (i, 0)),
            pl.BlockSpec((de, H), lambda i: (0, 0)),
            pl.BlockSpec((1, H), lambda i: (0, 0)),
            pl.BlockSpec((H, H * HP), lambda i: (0, 0)),
            pl.BlockSpec((1, H * HP), lambda i: (0, 0)),
        ],
        out_specs=[
            pl.BlockSpec((BE, HP), lambda i: (i, 0)),
            pl.BlockSpec((BE, HP), lambda i: (i, 0)),
        ],
        out_shape=[
            jax.ShapeDtypeStruct((e_pad, HP), jnp.float32),
            jax.ShapeDtypeStruct((e_pad, HP), jnp.float32),
        ],
    )(ea, xf, xr, w1, b1, w2p.astype(jnp.bfloat16), b2p)
    return of, orv


def _gather_body(table_ids, rpws, uniq, *refs):
    ntab = max(table_ids) + 1
    nset = len(rpws)
    tables = refs[:ntab]
    idxs = refs[ntab:ntab + nset]
    outs = refs[ntab + nset:ntab + 2 * nset]
    nu = len(uniq)
    vbuf = refs[ntab + 2 * nset]
    ibufs = refs[ntab + 2 * nset + 1:ntab + 2 * nset + 1 + nu]
    sem = refs[-1]
    cid = lax.axis_index("c")
    sid = lax.axis_index("s")
    wid = sid * NC + cid
    for k in range(nset):
        rpw = rpws[k]
        ib = ibufs[uniq.index(rpw)]
        base = wid * rpw
        pltpu.sync_copy(idxs[k].at[pl.ds(base, rpw)], ib)
        pltpu.async_copy(tables[table_ids[k]].at[ib],
                         vbuf.at[pl.ds(0, rpw)], sem).wait()
        pltpu.sync_copy(vbuf.at[pl.ds(0, rpw)], outs[k].at[pl.ds(base, rpw)])


def _sc_gather(tables, table_ids, idxs):
    rpws = tuple(ix.shape[0] // (NC * NS) for ix in idxs)
    uniq = sorted(set(rpws), reverse=True)
    mesh = plsc.VectorSubcoreMesh(core_axis_name="c", subcore_axis_name="s",
                                  num_cores=NC, num_subcores=NS)
    outs = pl.kernel(
        functools.partial(_gather_body, table_ids, rpws, uniq),
        out_type=[jax.ShapeDtypeStruct((ix.shape[0], HP), jnp.float32)
                  for ix in idxs],
        mesh=mesh,
        scratch_types=(
            [pltpu.VMEM((uniq[0], HP), jnp.float32)]
            + [pltpu.VMEM((r,), jnp.int32) for r in uniq]
            + [pltpu.SemaphoreType.DMA]),
    )(*tables, *idxs)
    return outs


def _scatter_body(eps, *refs):
    msgs = refs[0:6]
    idxs = refs[6:12]
    zeros = refs[12]
    out = refs[13]
    vbuf = refs[14]
    ibuf = refs[15]
    sh = refs[16]
    cid = lax.axis_index("c")
    sid = lax.axis_index("s")
    pltpu.sync_copy(zeros, sh.at[pl.ds(sid * STRIPE, STRIPE)])
    plsc.subcore_barrier()
    for k in range(6):
        nch = eps[k] // CH
        for j in range(-(-nch // NS)):
            c = sid + NS * j

            @pl.when(jnp.logical_and(cid == SET_CORE[k], c < nch))
            def _():
                pltpu.sync_copy(idxs[k].at[c], ibuf)
                pltpu.sync_copy(msgs[k].at[pl.ds(c * CH, CH)], vbuf)
                pltpu.sync_copy(vbuf, sh.at[ibuf], add=True)
    plsc.subcore_barrier()
    pltpu.sync_copy(sh.at[pl.ds(sid * STRIPE, STRIPE)],
                    out.at[cid].at[pl.ds(sid * STRIPE, STRIPE)])


def _sc_scatter(msgs, idxs, width):
    eps = tuple(m.shape[0] for m in msgs)
    mesh = plsc.VectorSubcoreMesh(core_axis_name="c", subcore_axis_name="s",
                                  num_cores=NC, num_subcores=NS)
    zeros = jnp.zeros((STRIPE, width), jnp.float32)
    out = pl.kernel(
        functools.partial(_scatter_body, eps),
        out_type=jax.ShapeDtypeStruct((NC, SH, width), jnp.float32),
        mesh=mesh,
        scratch_types=[
            pltpu.VMEM((CH, width), jnp.float32),
            pltpu.VMEM((CH,), jnp.int32),
            pltpu.VMEM_SHARED((SH, width), jnp.float32),
        ],
    )(*msgs, *[ix.reshape(-1, CH) for ix in idxs], zeros)
    return out


def _prep_idx(idx, e_pad, off):
    return (jnp.pad(idx, (0, e_pad - idx.shape[0])) + off).astype(jnp.int32)


def _pad_rows(x, e_pad):
    return jnp.pad(x, ((0, e_pad - x.shape[0]), (0, 0)))


def kernel(x_ap, x_user, x_target, ea_s, ea_tx, ea_rx, params, ei_s, ei_tx,
           ei_rx):
    h_ap = _encode(x_ap, params["ap_in"], params["ln"]["ap"], NP_AP)
    h_user = _encode(x_user, params["user_in"], params["ln"]["user"], NP_USER)
    h_tgt = _encode(x_target, params["tgt_in"], params["ln"]["tgt"], NP_TGT)

    e_s, e_tx, e_rx = ea_s.shape[0], ea_tx.shape[0], ea_rx.shape[0]
    ep_s = (e_s + BE - 1) // BE * BE
    ep_tx = (e_tx + BE - 1) // BE * BE
    ep_rx = (e_rx + BE - 1) // BE * BE
    ea_s_p = _pad_rows(ea_s, ep_s)
    ea_tx_p = _pad_rows(ea_tx, ep_tx)
    ea_rx_p = _pad_rows(ea_rx, ep_rx)

    n_ap, n_user, n_tgt = x_ap.shape[0], x_user.shape[0], x_target.shape[0]

    idx2ds = (
        _prep_idx(ei_s[1], ep_s, SET_OFF[0]),
        _prep_idx(ei_s[0], ep_s, SET_OFF[1]),
        _prep_idx(ei_tx[0], ep_tx, SET_OFF[2]),
        _prep_idx(ei_rx[0], ep_rx, SET_OFF[3]),
        _prep_idx(ei_tx[1], ep_tx, SET_OFF[4]),
        _prep_idx(ei_rx[1], ep_rx, SET_OFF[5]),
    )

    def ones_masked(e_pad, e_real):
        return jnp.where(jnp.arange(e_pad)[:, None] < e_real,
                         jnp.float32(1), jnp.float32(0)) * jnp.ones((1, HP),
                                                                    jnp.float32)
    cnt = _sc_scatter(
        (ones_masked(ep_s, e_s), ones_masked(ep_s, e_s),
         ones_masked(ep_tx, e_tx), ones_masked(ep_rx, e_rx),
         ones_masked(ep_tx, e_tx), ones_masked(ep_rx, e_rx)),
        idx2ds, HP)
    inv = 1.0 / jnp.maximum(cnt, 1.0)

    gidx = (
        _prep_idx(ei_s[0], ep_s, 0), _prep_idx(ei_s[1], ep_s, 0),
        _prep_idx(ei_tx[0], ep_tx, 0), _prep_idx(ei_tx[1], ep_tx, 0),
        _prep_idx(ei_rx[0], ep_rx, 0), _prep_idx(ei_rx[1], ep_rx, 0),
    )

    for lp in params["layers"]:
        c = lp["conv"]
        (xs_ap, xs_user, xtx_ap, xtx_tgt, xrx_ap, xrx_tgt) = _sc_gather(
            (h_ap, h_user, h_tgt), (0, 1, 0, 2, 0, 2), gidx)

        m_s_f, m_s_r = _fused_msgs(ea_s_p, xs_ap, xs_user, lp["net_serv"], e_s)
        m_tx_f, m_tx_r = _fused_msgs(ea_tx_p, xtx_ap, xtx_tgt,
                                     lp["net_sens"], e_tx)
        m_rx_f, m_rx_r = _fused_msgs(ea_rx_p, xrx_ap, xrx_tgt,
                                     lp["net_sens"], e_rx)

        p = _sc_scatter((m_s_f, m_s_r, m_tx_r, m_rx_r, m_tx_f, m_rx_f),
                        idx2ds, HP)
        h_user = _combine(h_user, [c["serves"]["root"]],
                          [c["serves"]["bias"]], p, inv, (0,), NP_USER)
        h_ap = _combine(h_ap,
                        [c["rev_served"]["root"], c["rev_tx"]["root"],
                         c["rev_rx"]["root"]],
                        [c["rev_served"]["bias"], c["rev_tx"]["bias"],
                         c["rev_rx"]["bias"]], p, inv, (1, 2, 3), NP_AP)
        h_tgt = _combine(h_tgt, [c["tx"]["root"], c["rx"]["root"]],
                         [c["tx"]["bias"], c["rx"]["bias"]], p, inv, (4, 5),
                         NP_TGT)

    ga, gu = _sc_gather((h_ap, h_user), (0, 1), (gidx[0], gidx[1]))
    x_logit = _head(params["edge_head"], ga, gu, ea_s_p)[:e_s, :1]
    tau_logit = _head(params["ap_head"], h_ap)[:n_ap, :1]
    s_logit = _head(params["tgt_head"], h_tgt)[:n_tgt, :1]
    return x_logit, tau_logit, s_logit
